# baseline (speedup 1.0000x reference)
import functools

import jax
import jax.numpy as jnp
from jax import lax
from jax.experimental import pallas as pl
from jax.experimental.pallas import tpu as pltpu
from jax.experimental.pallas import tpu_sc as plsc

N = 10000
NP = 10240
E = 320000
NC, NS = 2, 16
NW = NC * NS
K = 120
EPW = 10080
E_PAD = NW * EPW
CH = EPW // K
RPT = NP // NS

DOUTS = (256, 128, 64, 32)
DINS = (128, 384, 512, 576)
BR = 512


def _make_spmm(wc):
    mesh = plsc.VectorSubcoreMesh(core_axis_name="c", subcore_axis_name="s")

    nbuf = 3
    G = 6
    NG = CH // G

    @functools.partial(
        pl.kernel,
        out_type=jax.ShapeDtypeStruct((NC, NP, wc), jnp.float32),
        mesh=mesh,
        scratch_types=[
            pltpu.VMEM_SHARED((NP, wc), jnp.float32),
        ] + [pltpu.VMEM((K, wc), jnp.float32) for _ in range(nbuf)]
          + [pltpu.VMEM((G, K), jnp.int32) for _ in range(2)]
          + [pltpu.SemaphoreType.DMA for _ in range(2 * nbuf)],
    )
    def spmm(table, srcp, dstp, zeros, out, acc, *bufs):
        rows = bufs[:nbuf]
        sslab, dslab = bufs[nbuf:nbuf + 2]
        semg = bufs[nbuf + 2:nbuf + 2 + nbuf]
        sems = bufs[nbuf + 2 + nbuf:]
        cid = lax.axis_index("c")
        sid = lax.axis_index("s")
        wid = cid * NS + sid

        pltpu.sync_copy(zeros, acc.at[pl.ds(sid * RPT, RPT)])
        plsc.subcore_barrier()

        @pl.loop(0, NG)
        def _(g):
            @pl.when(g > 0)
            def _drain():
                for b in range(nbuf):
                    pltpu.make_async_copy(rows[b], acc.at[dslab.at[b]],
                                          sems[b]).wait()
            row = wid * NG + g
            pltpu.sync_copy(srcp.at[row], sslab)
            pltpu.sync_copy(dstp.at[row], dslab)
            for j in range(G):
                b = j % nbuf
                if j >= nbuf:
                    pltpu.make_async_copy(rows[b], acc.at[dslab.at[b]],
                                          sems[b]).wait()
                pltpu.async_copy(table.at[sslab.at[j]], rows[b], semg[b])
                if j >= nbuf - 1:
                    jj = j - (nbuf - 1)
                    bb = jj % nbuf
                    pltpu.make_async_copy(table.at[sslab.at[jj]], rows[bb],
                                          semg[bb]).wait()
                    pltpu.async_copy(rows[bb], acc.at[dslab.at[jj]],
                                     sems[bb], add=True)
            for j in range(G - (nbuf - 1), G):
                b = j % nbuf
                pltpu.make_async_copy(table.at[sslab.at[j]], rows[b],
                                      semg[b]).wait()
                pltpu.async_copy(rows[b], acc.at[dslab.at[j]], sems[b],
                                 add=True)

        for b in range(nbuf):
            pltpu.make_async_copy(rows[b], acc.at[dslab.at[b]],
                                  sems[b]).wait()
        plsc.subcore_barrier()
        pltpu.sync_copy(acc.at[pl.ds(sid * RPT, RPT)],
                        out.at[cid].at[pl.ds(sid * RPT, RPT)])

    return spmm


_SPMM = {wc: _make_spmm(wc) for wc in (128,)}


def _leaky(v):
    return jnp.where(v >= 0, v, 0.01 * v)


def _dense_body(nx, dout, nch, wc, nprev, dout_prev, refs):
    xs = list(refs[:nx])
    r = nx
    if nprev:
        ps = refs[r:r + nprev]
        qp = refs[r + nprev]
        r += nprev + 1
    wsrc, wlin, wpos, bpos = refs[r:r + 4]
    outs = refs[r + 4:]
    t_ref, q_ref = outs[0], outs[1]
    if nprev:
        full = jnp.concatenate([pr[0] + pr[1] for pr in ps], axis=1)
        denom = full[:, :dout_prev]
        num = full[:, dout_prev:2 * dout_prev]
        inv = 1.0 / (denom + 1e-16)
        y = _leaky(num * inv + qp[...] * (denom * inv))
        outs[2][...] = y
        xs = xs + [y]
    pos = xs[0][:, :3]
    p = jnp.dot(pos, wpos[...], preferred_element_type=jnp.float32)
    b = p
    g = -p
    off = 0
    for xr in xs:
        xb = xr if isinstance(xr, jax.Array) else xr[...]
        w = xb.shape[1]
        b = b + jnp.dot(xb, wsrc[off:off + w, :],
                        preferred_element_type=jnp.float32)
        g = g + jnp.dot(xb, wlin[off:off + w, :],
                        preferred_element_type=jnp.float32)
        off += w
    c = jnp.exp(-b)
    t = jnp.concatenate([c, c * g], axis=1)
    if t.shape[1] < nch * wc:
        t = jnp.concatenate(
            [t, jnp.zeros((t.shape[0], nch * wc - t.shape[1]), t.dtype)],
            axis=1)
    for ch in range(nch):
        t_ref[ch] = t[:, ch * wc:(ch + 1) * wc]
    q_ref[...] = p + bpos[...]


def _dense_call(xs, wsrc_t, wlin_t, wpos_t, bpos, dout, wc,
                prev=None):
    nch = max(1, (2 * dout) // wc)
    nx = len(xs)
    din = sum(x.shape[1] for x in xs)
    grid = (NP // BR,)
    if prev is None:
        nprev, dout_prev, prev_args = 0, 0, []
        prev_specs = []
        y_shape, y_specs = [], []
    else:
        ps, qp, dout_prev = prev
        nprev = len(ps)
        din += dout_prev
        prev_args = list(ps) + [qp]
        prev_specs = (
            [pl.BlockSpec((NC, BR, wc), lambda i: (0, i, 0)) for _ in ps]
            + [pl.BlockSpec((BR, dout_prev), lambda i: (i, 0))]
        )
        y_shape = [jax.ShapeDtypeStruct((NP, dout_prev), jnp.float32)]
        y_specs = [pl.BlockSpec((BR, dout_prev), lambda i: (i, 0))]
    in_specs = (
        [pl.BlockSpec((BR, x.shape[1]), lambda i: (i, 0)) for x in xs]
        + prev_specs
        + [pl.BlockSpec((din, dout), lambda i: (0, 0)),
           pl.BlockSpec((din, dout), lambda i: (0, 0)),
           pl.BlockSpec((3, dout), lambda i: (0, 0)),
           pl.BlockSpec((1, dout), lambda i: (0, 0))]
    )
    out_specs = [
        pl.BlockSpec((nch, BR, wc), lambda i: (0, i, 0)),
        pl.BlockSpec((BR, dout), lambda i: (i, 0)),
    ] + y_specs
    out_shape = [
        jax.ShapeDtypeStruct((nch, NP, wc), jnp.float32),
        jax.ShapeDtypeStruct((NP, dout), jnp.float32),
    ] + y_shape
    body = lambda *refs: _dense_body(nx, dout, nch, wc, nprev, dout_prev,
                                     refs)
    return pl.pallas_call(
        body, grid=grid, in_specs=in_specs, out_specs=out_specs,
        out_shape=out_shape,
    )(*xs, *prev_args, wsrc_t, wlin_t, wpos_t, bpos)


def _final_body(refs):
    (x0, y1, y2, y3, p4, q4, m1w, m1b, m2w, m2b, out_ref) = refs
    full = p4[0] + p4[1]
    dout = 32
    denom = full[:, :dout]
    num = full[:, dout:2 * dout]
    inv = 1.0 / (denom + 1e-16)
    y4 = _leaky(num * inv + q4[...] * (denom * inv))
    h = jnp.dot(x0[...], m1w[0:128, :], preferred_element_type=jnp.float32)
    h = h + jnp.dot(y1[...], m1w[128:384, :], preferred_element_type=jnp.float32)
    h = h + jnp.dot(y2[...], m1w[384:512, :], preferred_element_type=jnp.float32)
    h = h + jnp.dot(y3[...], m1w[512:576, :], preferred_element_type=jnp.float32)
    h = h + jnp.dot(y4, m1w[576:608, :], preferred_element_type=jnp.float32)
    h = h + m1b[...]
    o = jnp.dot(h, m2w[...], preferred_element_type=jnp.float32) + m2b[...]
    out_ref[...] = o


def _final_call(x0, y1, y2, y3, p4, q4, m1w_t, m1b, m2w_t, m2b):
    grid = (NP // BR,)
    in_specs = [
        pl.BlockSpec((BR, 128), lambda i: (i, 0)),
        pl.BlockSpec((BR, 256), lambda i: (i, 0)),
        pl.BlockSpec((BR, 128), lambda i: (i, 0)),
        pl.BlockSpec((BR, 64), lambda i: (i, 0)),
        pl.BlockSpec((NC, BR, 128), lambda i: (0, i, 0)),
        pl.BlockSpec((BR, 32), lambda i: (i, 0)),
        pl.BlockSpec((608, 64), lambda i: (0, 0)),
        pl.BlockSpec((1, 64), lambda i: (0, 0)),
        pl.BlockSpec((64, 128), lambda i: (0, 0)),
        pl.BlockSpec((1, 128), lambda i: (0, 0)),
    ]
    out_specs = pl.BlockSpec((BR, 128), lambda i: (i, 0))
    return pl.pallas_call(
        lambda *refs: _final_body(refs), grid=grid,
        in_specs=in_specs, out_specs=out_specs,
        out_shape=jax.ShapeDtypeStruct((NP, 128), jnp.float32),
    )(x0, y1, y2, y3, p4, q4, m1w_t, m1b, m2w_t, m2b)


def kernel(x, edge_index,
           c1_W_lin, c1_W_src, c1_W_dst, c1_W_pos, c1_b_pos,
           c2_W_lin, c2_W_src, c2_W_dst, c2_W_pos, c2_b_pos,
           c3_W_lin, c3_W_src, c3_W_dst, c3_W_pos, c3_b_pos,
           c4_W_lin, c4_W_src, c4_W_dst, c4_W_pos, c4_b_pos,
           m1_W, m1_b, m2_W, m2_b):
    del c1_W_dst, c2_W_dst, c3_W_dst, c4_W_dst

    x0 = jnp.zeros((NP, 128), jnp.float32).at[:N].set(x)
    srcp = jnp.concatenate(
        [edge_index[0], jnp.arange(E_PAD - E, dtype=jnp.int32) % N])
    dstp = jnp.concatenate(
        [edge_index[1],
         N + (jnp.arange(E_PAD - E, dtype=jnp.int32) % (NP - N))])
    nslab = E_PAD // (6 * K)
    srcp = srcp.reshape(nslab, 6, K)
    dstp = dstp.reshape(nslab, 6, K)
    z128 = jnp.zeros((RPT, 128), jnp.float32)

    ws = [
        (c1_W_lin, c1_W_src, c1_W_pos, c1_b_pos),
        (c2_W_lin, c2_W_src, c2_W_pos, c2_b_pos),
        (c3_W_lin, c3_W_src, c3_W_pos, c3_b_pos),
        (c4_W_lin, c4_W_src, c4_W_pos, c4_b_pos),
    ]

    xs = [x0]
    prev = None
    last = None
    for li, (dout, (wlin, wsrc, wpos, bpos)) in enumerate(zip(DOUTS, ws)):
        wc = 128
        res = _dense_call(xs, wsrc.T, wlin.T, wpos.T, bpos[None, :],
                          dout, wc, prev=prev)
        if prev is None:
            t, q = res
        else:
            t, q, y_prev = res
            xs.append(y_prev)
        nch = max(1, (2 * dout) // wc)
        ps = [_SPMM[wc](t[ch], srcp, dstp, z128) for ch in range(nch)]
        prev = (ps, q, dout)
        last = (ps[0], q)

    p4, q4 = last
    out = _final_call(xs[0], xs[1], xs[2], xs[3], p4, q4,
                      m1_W.T, m1_b[None, :], m2_W.T, m2_b[None, :])
    return out[:N]


# speedup vs baseline: 1.0286x; 1.0286x over previous
import functools

import jax
import jax.numpy as jnp
from jax import lax
from jax.experimental import pallas as pl
from jax.experimental.pallas import tpu as pltpu
from jax.experimental.pallas import tpu_sc as plsc

N = 10000
NP = 10240
E = 320000
NC, NS = 2, 16
NW = NC * NS
K = 120
EPW = 10080
E_PAD = NW * EPW
CH = EPW // K
RPT = NP // NS

DOUTS = (256, 128, 64, 32)
DINS = (128, 384, 512, 576)
BR = 512


def _make_spmm(wc):
    mesh = plsc.VectorSubcoreMesh(core_axis_name="c", subcore_axis_name="s")

    nbuf = 3
    G = 6
    NG = CH // G

    @functools.partial(
        pl.kernel,
        out_type=jax.ShapeDtypeStruct((NC, NP, wc), jnp.float32),
        mesh=mesh,
        scratch_types=[
            pltpu.VMEM_SHARED((NP, wc), jnp.float32),
        ] + [pltpu.VMEM((K, wc), jnp.float32) for _ in range(nbuf)]
          + [pltpu.VMEM((G, K), jnp.int32) for _ in range(2)]
          + [pltpu.SemaphoreType.DMA for _ in range(2 * nbuf)],
    )
    def spmm(table, srcp, dstp, out, acc, *bufs):
        rows = bufs[:nbuf]
        sslab, dslab = bufs[nbuf:nbuf + 2]
        semg = bufs[nbuf + 2:nbuf + 2 + nbuf]
        sems = bufs[nbuf + 2 + nbuf:]
        cid = lax.axis_index("c")
        sid = lax.axis_index("s")
        wid = cid * NS + sid

        z16 = jnp.zeros((16,), jnp.float32)

        @pl.loop(0, K)
        def _z(r):
            for c in range(wc // 16):
                rows[0][r, pl.ds(c * 16, 16)] = z16

        for t in range(RPT // K):
            pltpu.sync_copy(rows[0], acc.at[pl.ds(sid * RPT + t * K, K)])
        rem = RPT % K
        if rem:
            pltpu.sync_copy(rows[0].at[pl.ds(0, rem)],
                            acc.at[pl.ds(sid * RPT + (RPT // K) * K, rem)])
        plsc.subcore_barrier()

        @pl.loop(0, NG)
        def _(g):
            @pl.when(g > 0)
            def _drain():
                for b in range(nbuf):
                    pltpu.make_async_copy(rows[b], acc.at[dslab.at[b]],
                                          sems[b]).wait()
            row = wid * NG + g
            pltpu.sync_copy(srcp.at[row], sslab)
            pltpu.sync_copy(dstp.at[row], dslab)
            for j in range(G):
                b = j % nbuf
                if j >= nbuf:
                    pltpu.make_async_copy(rows[b], acc.at[dslab.at[b]],
                                          sems[b]).wait()
                pltpu.async_copy(table.at[sslab.at[j]], rows[b], semg[b])
                if j >= nbuf - 1:
                    jj = j - (nbuf - 1)
                    bb = jj % nbuf
                    pltpu.make_async_copy(table.at[sslab.at[jj]], rows[bb],
                                          semg[bb]).wait()
                    pltpu.async_copy(rows[bb], acc.at[dslab.at[jj]],
                                     sems[bb], add=True)
            for j in range(G - (nbuf - 1), G):
                b = j % nbuf
                pltpu.make_async_copy(table.at[sslab.at[j]], rows[b],
                                      semg[b]).wait()
                pltpu.async_copy(rows[b], acc.at[dslab.at[j]], sems[b],
                                 add=True)

        for b in range(nbuf):
            pltpu.make_async_copy(rows[b], acc.at[dslab.at[b]],
                                  sems[b]).wait()
        plsc.subcore_barrier()
        pltpu.sync_copy(acc.at[pl.ds(sid * RPT, RPT)],
                        out.at[cid].at[pl.ds(sid * RPT, RPT)])

    return spmm


_SPMM = {wc: _make_spmm(wc) for wc in (128,)}


def _leaky(v):
    return jnp.where(v >= 0, v, 0.01 * v)


def _dense_body(nx, dout, nch, wc, nprev, dout_prev, refs):
    xs = list(refs[:nx])
    r = nx
    if nprev:
        ps = refs[r:r + nprev]
        qp = refs[r + nprev]
        r += nprev + 1
    wsrc, wlin, wpos, bpos = refs[r:r + 4]
    outs = refs[r + 4:]
    t_ref, q_ref = outs[0], outs[1]
    if nprev:
        full = jnp.concatenate([pr[0] + pr[1] for pr in ps], axis=1)
        denom = full[:, :dout_prev]
        num = full[:, dout_prev:2 * dout_prev]
        inv = 1.0 / (denom + 1e-16)
        y = _leaky(num * inv + qp[...] * (denom * inv))
        outs[2][...] = y
        xs = xs + [y]
    pos = xs[0][:, :3]
    p = jnp.dot(pos, wpos[...], preferred_element_type=jnp.float32)
    b = p
    g = -p
    off = 0
    for xr in xs:
        xb = xr if isinstance(xr, jax.Array) else xr[...]
        w = xb.shape[1]
        b = b + jnp.dot(xb, wsrc[off:off + w, :],
                        preferred_element_type=jnp.float32)
        g = g + jnp.dot(xb, wlin[off:off + w, :],
                        preferred_element_type=jnp.float32)
        off += w
    c = jnp.exp(-b)
    t = jnp.concatenate([c, c * g], axis=1)
    if t.shape[1] < nch * wc:
        t = jnp.concatenate(
            [t, jnp.zeros((t.shape[0], nch * wc - t.shape[1]), t.dtype)],
            axis=1)
    for ch in range(nch):
        t_ref[ch] = t[:, ch * wc:(ch + 1) * wc]
    q_ref[...] = p + bpos[...]


def _dense_call(xs, wsrc_t, wlin_t, wpos_t, bpos, dout, wc,
                prev=None):
    nch = max(1, (2 * dout) // wc)
    nx = len(xs)
    din = sum(x.shape[1] for x in xs)
    grid = (NP // BR,)
    if prev is None:
        nprev, dout_prev, prev_args = 0, 0, []
        prev_specs = []
        y_shape, y_specs = [], []
    else:
        ps, qp, dout_prev = prev
        nprev = len(ps)
        din += dout_prev
        prev_args = list(ps) + [qp]
        prev_specs = (
            [pl.BlockSpec((NC, BR, wc), lambda i: (0, i, 0)) for _ in ps]
            + [pl.BlockSpec((BR, dout_prev), lambda i: (i, 0))]
        )
        y_shape = [jax.ShapeDtypeStruct((NP, dout_prev), jnp.float32)]
        y_specs = [pl.BlockSpec((BR, dout_prev), lambda i: (i, 0))]
    in_specs = (
        [pl.BlockSpec((BR, x.shape[1]), lambda i: (i, 0)) for x in xs]
        + prev_specs
        + [pl.BlockSpec((din, dout), lambda i: (0, 0)),
           pl.BlockSpec((din, dout), lambda i: (0, 0)),
           pl.BlockSpec((3, dout), lambda i: (0, 0)),
           pl.BlockSpec((1, dout), lambda i: (0, 0))]
    )
    out_specs = [
        pl.BlockSpec((nch, BR, wc), lambda i: (0, i, 0)),
        pl.BlockSpec((BR, dout), lambda i: (i, 0)),
    ] + y_specs
    out_shape = [
        jax.ShapeDtypeStruct((nch, NP, wc), jnp.float32),
        jax.ShapeDtypeStruct((NP, dout), jnp.float32),
    ] + y_shape
    body = lambda *refs: _dense_body(nx, dout, nch, wc, nprev, dout_prev,
                                     refs)
    return pl.pallas_call(
        body, grid=grid, in_specs=in_specs, out_specs=out_specs,
        out_shape=out_shape,
    )(*xs, *prev_args, wsrc_t, wlin_t, wpos_t, bpos)


def _final_body(refs):
    (x0, y1, y2, y3, p4, q4, m1w, m1b, m2w, m2b, out_ref) = refs
    full = p4[0] + p4[1]
    dout = 32
    denom = full[:, :dout]
    num = full[:, dout:2 * dout]
    inv = 1.0 / (denom + 1e-16)
    y4 = _leaky(num * inv + q4[...] * (denom * inv))
    h = jnp.dot(x0[...], m1w[0:128, :], preferred_element_type=jnp.float32)
    h = h + jnp.dot(y1[...], m1w[128:384, :], preferred_element_type=jnp.float32)
    h = h + jnp.dot(y2[...], m1w[384:512, :], preferred_element_type=jnp.float32)
    h = h + jnp.dot(y3[...], m1w[512:576, :], preferred_element_type=jnp.float32)
    h = h + jnp.dot(y4, m1w[576:608, :], preferred_element_type=jnp.float32)
    h = h + m1b[...]
    o = jnp.dot(h, m2w[...], preferred_element_type=jnp.float32) + m2b[...]
    out_ref[...] = o


def _final_call(x0, y1, y2, y3, p4, q4, m1w_t, m1b, m2w_t, m2b):
    grid = (NP // BR,)
    in_specs = [
        pl.BlockSpec((BR, 128), lambda i: (i, 0)),
        pl.BlockSpec((BR, 256), lambda i: (i, 0)),
        pl.BlockSpec((BR, 128), lambda i: (i, 0)),
        pl.BlockSpec((BR, 64), lambda i: (i, 0)),
        pl.BlockSpec((NC, BR, 128), lambda i: (0, i, 0)),
        pl.BlockSpec((BR, 32), lambda i: (i, 0)),
        pl.BlockSpec((608, 64), lambda i: (0, 0)),
        pl.BlockSpec((1, 64), lambda i: (0, 0)),
        pl.BlockSpec((64, 128), lambda i: (0, 0)),
        pl.BlockSpec((1, 128), lambda i: (0, 0)),
    ]
    out_specs = pl.BlockSpec((BR, 128), lambda i: (i, 0))
    return pl.pallas_call(
        lambda *refs: _final_body(refs), grid=grid,
        in_specs=in_specs, out_specs=out_specs,
        out_shape=jax.ShapeDtypeStruct((NP, 128), jnp.float32),
    )(x0, y1, y2, y3, p4, q4, m1w_t, m1b, m2w_t, m2b)


def kernel(x, edge_index,
           c1_W_lin, c1_W_src, c1_W_dst, c1_W_pos, c1_b_pos,
           c2_W_lin, c2_W_src, c2_W_dst, c2_W_pos, c2_b_pos,
           c3_W_lin, c3_W_src, c3_W_dst, c3_W_pos, c3_b_pos,
           c4_W_lin, c4_W_src, c4_W_dst, c4_W_pos, c4_b_pos,
           m1_W, m1_b, m2_W, m2_b):
    del c1_W_dst, c2_W_dst, c3_W_dst, c4_W_dst

    x0 = jnp.zeros((NP, 128), jnp.float32).at[:N].set(x)
    srcp = jnp.concatenate(
        [edge_index[0], jnp.arange(E_PAD - E, dtype=jnp.int32) % N])
    dstp = jnp.concatenate(
        [edge_index[1],
         N + (jnp.arange(E_PAD - E, dtype=jnp.int32) % (NP - N))])
    nslab = E_PAD // (6 * K)
    srcp = srcp.reshape(nslab, 6, K)
    dstp = dstp.reshape(nslab, 6, K)

    ws = [
        (c1_W_lin, c1_W_src, c1_W_pos, c1_b_pos),
        (c2_W_lin, c2_W_src, c2_W_pos, c2_b_pos),
        (c3_W_lin, c3_W_src, c3_W_pos, c3_b_pos),
        (c4_W_lin, c4_W_src, c4_W_pos, c4_b_pos),
    ]

    xs = [x0]
    prev = None
    last = None
    for li, (dout, (wlin, wsrc, wpos, bpos)) in enumerate(zip(DOUTS, ws)):
        wc = 128
        res = _dense_call(xs, wsrc.T, wlin.T, wpos.T, bpos[None, :],
                          dout, wc, prev=prev)
        if prev is None:
            t, q = res
        else:
            t, q, y_prev = res
            xs.append(y_prev)
        nch = max(1, (2 * dout) // wc)
        ps = [_SPMM[wc](t[ch], srcp, dstp) for ch in range(nch)]
        prev = (ps, q, dout)
        last = (ps[0], q)

    p4, q4 = last
    out = _final_call(xs[0], xs[1], xs[2], xs[3], p4, q4,
                      m1_W.T, m1_b[None, :], m2_W.T, m2_b[None, :])
    return out[:N]
